# baseline (speedup 1.0000x reference)
import functools

import jax
import jax.numpy as jnp
from jax import lax
from jax.experimental import pallas as pl
from jax.experimental.pallas import tpu as pltpu
from jax.experimental.pallas import tpu_sc as plsc

NUM_CORES = 2
NUM_SUBCORES = 16
NUM_WORKERS = NUM_CORES * NUM_SUBCORES
NBUF = 8
SBUF = 4


@functools.partial(jax.jit, static_argnames=("chunk",))
def _sc_gather(idx_flat, W, chunk=80):
    B = idx_flat.shape[0]
    D = W.shape[1]
    b_per_w = B // NUM_WORKERS
    n_chunks = b_per_w // chunk
    assert b_per_w % chunk == 0 and chunk % 8 == 0
    assert n_chunks % NBUF == 0 and NBUF % SBUF == 0
    n_laps = n_chunks // NBUF
    assert n_laps >= 3

    mesh = plsc.VectorSubcoreMesh(
        core_axis_name="c", subcore_axis_name="s",
        num_cores=NUM_CORES, num_subcores=NUM_SUBCORES,
    )

    scratch = (
        [pltpu.VMEM((b_per_w,), jnp.int32)]
        + [pltpu.VMEM_SHARED((NUM_SUBCORES, SBUF, chunk, D), jnp.float32)]
        + [pltpu.VMEM((chunk, D), jnp.float32) for _ in range(NBUF)]
        + [pltpu.SemaphoreType.DMA for _ in range(2 * NBUF + SBUF)]
    )

    @functools.partial(
        pl.kernel,
        mesh=mesh,
        out_type=jax.ShapeDtypeStruct((B, D), jnp.float32),
        scratch_types=scratch,
    )
    def k(idx_hbm, table_hbm, out_hbm, idx_v, spmem_v, *refs):
        rows_v = refs[:NBUF]
        gsem = refs[NBUF:2 * NBUF]
        wsem = refs[2 * NBUF:3 * NBUF]
        dsem = refs[3 * NBUF:]

        sid = lax.axis_index("s")
        wid = sid * NUM_CORES + lax.axis_index("c")
        base = wid * b_per_w

        pltpu.sync_copy(idx_hbm.at[pl.ds(base, b_per_w)], idx_v)

        def start_gather(c, b):
            pltpu.async_copy(
                table_hbm.at[idx_v.at[pl.ds(c * chunk, chunk)]],
                rows_v[b], gsem[b])

        def wait_gather(c, b):
            pltpu.make_async_copy(
                table_hbm.at[idx_v.at[pl.ds(c * chunk, chunk)]],
                rows_v[b], gsem[b]).wait()

        def hop_to_spmem(b):
            s = b % SBUF
            pltpu.async_copy(rows_v[b], spmem_v.at[sid, s], wsem[b])
            pltpu.make_async_copy(
                rows_v[b], spmem_v.at[sid, s], wsem[b]).wait()

        def start_dma_out(c, b):
            off = base + c * chunk
            pltpu.async_copy(
                spmem_v.at[sid, b % SBUF],
                out_hbm.at[pl.ds(off, chunk)], dsem[b % SBUF])

        def wait_dma_out(c, b):
            off = base + c * chunk
            pltpu.make_async_copy(
                spmem_v.at[sid, b % SBUF],
                out_hbm.at[pl.ds(off, chunk)], dsem[b % SBUF]).wait()

        for b in range(NBUF):
            start_gather(b, b)

        for b in range(NBUF):
            wait_gather(b, b)
            if b >= SBUF:
                wait_dma_out(b - SBUF, b - SBUF)
            hop_to_spmem(b)
            start_gather(b + NBUF, b)
            start_dma_out(b, b)

        def body(j, carry):
            i = j + 1
            for b in range(NBUF):
                c = i * NBUF + b
                wait_gather(c, b)
                wait_dma_out(c - SBUF, b - SBUF)
                hop_to_spmem(b)
                start_gather(c + NBUF, b)
                start_dma_out(c, b)
            return carry

        lax.fori_loop(0, n_laps - 2, body, 0)

        for b in range(NBUF):
            c = n_chunks - NBUF + b
            wait_gather(c, b)
            wait_dma_out(c - SBUF, b - SBUF)
            hop_to_spmem(b)
            start_dma_out(c, b)

        for b in range(NBUF - SBUF, NBUF):
            c = n_chunks - NBUF + b
            wait_dma_out(c, b)

    return k(idx_flat, W)


def kernel(token_ids, W):
    S, T = token_ids.shape
    D = W.shape[1]
    idx_t = jnp.swapaxes(token_ids, 0, 1).reshape(S * T).astype(jnp.int32)
    out = _sc_gather(idx_t, W)
    return jnp.transpose(out.reshape(T, S, D), (1, 0, 2))
